# Initial kernel scaffold; baseline (speedup 1.0000x reference)
import jax
import jax.numpy as jnp
from jax.experimental import pallas as pl


def kernel(x, hl1_edge_index, hl1_edge_weight, hl2_edge_index, hl2_edge_weight, W_in1, b_in1, fW1, W_in2, b_in2, fW2, W_out, b_out):
    raise NotImplementedError("write your pallas kernel here")


# speedup vs baseline: 3.0197x; 3.0197x over previous
import functools

import jax
import jax.numpy as jnp
from jax import lax
from jax.experimental import pallas as pl
from jax.experimental.pallas import tpu as pltpu, tpu_sc as plsc

_N = 10000
_NP = 10240
_E = 320000
_D = 128
_H = 64
_K = 10
_NT = 16
_EPT = _E // _NT
_C = 128
_NCH = (_EPT + _C - 1) // _C
_EPAD = _NCH * _C
_RPT = _NP // _NT
_RC = 128


def _proj_in_body(x_ref, w_ref, b_ref, o_ref):
    acc = jnp.dot(x_ref[...], w_ref[0], preferred_element_type=jnp.float32)
    o_ref[...] = acc + b_ref[0]


def _proj_in(x, W, b):
    bn = 1024
    grid = (2, _NP // bn)
    return pl.pallas_call(
        _proj_in_body,
        grid=grid,
        in_specs=[
            pl.BlockSpec((bn, _D), lambda c, i: (i, 0)),
            pl.BlockSpec((1, _D, _H), lambda c, i: (c, 0, 0)),
            pl.BlockSpec((1, 1, _H), lambda c, i: (c, 0, 0)),
        ],
        out_specs=pl.BlockSpec((bn, _H), lambda c, i: (c * (_NP // bn) + i, 0)),
        out_shape=jax.ShapeDtypeStruct((2 * _NP, _H), jnp.float32),
    )(x, W, b)


def _proj_out_body(o_ref, w_ref, b_ref, y_ref):
    z1 = jnp.dot(o_ref[0], w_ref[: _H], preferred_element_type=jnp.float32)
    z2 = jnp.dot(o_ref[1], w_ref[_H:], preferred_element_type=jnp.float32)
    y_ref[...] = z1 + z2 + b_ref[...][None, :]


def _proj_out(o, W_out, b_out):
    bn = 1000
    return pl.pallas_call(
        _proj_out_body,
        grid=(_N // bn,),
        in_specs=[
            pl.BlockSpec((2, bn, _H), lambda i: (0, i, 0)),
            pl.BlockSpec((_H * 2, _H), lambda i: (0, 0)),
            pl.BlockSpec((_H,), lambda i: (0,)),
        ],
        out_specs=pl.BlockSpec((bn, _H), lambda i: (i, 0)),
        out_shape=jax.ShapeDtypeStruct((_N, _H), jnp.float32),
    )(o, W_out, b_out)


def _sc_body(h_hbm, col_hbm, row_hbm, w_hbm, fw_hbm,
             out_hbm, ws_hbm,
             xsp, col_buf, row_buf, w_buf, gbuf, acc, tmp, fbuf):
    c = lax.axis_index("c")
    s = lax.axis_index("s")
    r0 = s * _RPT
    g0 = c * _NP + r0

    pltpu.sync_copy(col_hbm.at[c, s], col_buf)
    pltpu.sync_copy(row_hbm.at[c, s], row_buf)
    pltpu.sync_copy(w_hbm.at[c, s], w_buf)

    pltpu.sync_copy(fw_hbm.at[c], fbuf)
    fv = fbuf[...]
    ev = jnp.exp(fv)
    lanes = lax.iota(jnp.int32, 16)
    t = ev
    for sh in (1, 2, 4, 8):
        perm = jnp.bitwise_xor(lanes, sh)
        t = t + t.at[perm].get(mode="promise_in_bounds")
    f = ev / t

    f0 = f[0]
    for i in range(_RPT // _RC):
        pltpu.sync_copy(h_hbm.at[pl.ds(g0 + i * _RC, _RC)], tmp)
        pltpu.sync_copy(tmp, ws_hbm.at[pl.ds(g0 + i * _RC, _RC)])

        def ibody(r, _):
            for q in range(_H // 16):
                sl = pl.ds(q * 16, 16)
                acc[r, sl] = f0 * tmp[r, sl]
            return _
        lax.fori_loop(0, _RC, ibody, 0)
        pltpu.sync_copy(acc, out_hbm.at[c, pl.ds(r0 + i * _RC, _RC)])

    def hop(k, _):
        def zbody(r, _):
            for q in range(_H // 16):
                tmp[r, pl.ds(q * 16, 16)] = jnp.zeros((16,), jnp.float32)
            return _
        lax.fori_loop(0, _RC, zbody, 0)
        for i in range(_RPT // _RC):
            pltpu.sync_copy(tmp, xsp.at[pl.ds(r0 + i * _RC, _RC)])
        plsc.subcore_barrier()

        def chunk(j, _):
            pltpu.sync_copy(ws_hbm.at[col_buf.at[j]], gbuf)

            def mbody(g, _):
                wv = w_buf[j, pl.ds(g * 16, 16)]
                for lane in range(16):
                    w_s = wv[lane]
                    e = g * 16 + lane
                    for q in range(_H // 16):
                        sl = pl.ds(q * 16, 16)
                        gbuf[e, sl] = gbuf[e, sl] * w_s
                return _
            lax.fori_loop(0, _C // 16, mbody, 0)
            pltpu.sync_copy(gbuf, xsp.at[row_buf.at[j]], add=True)
            return _
        lax.fori_loop(0, _NCH, chunk, 0)
        plsc.subcore_barrier()

        fk = f.at[jnp.full((16,), k + 1, dtype=jnp.int32)].get(
            mode="promise_in_bounds")
        for i in range(_RPT // _RC):
            pltpu.sync_copy(xsp.at[pl.ds(r0 + i * _RC, _RC)], tmp)
            pltpu.sync_copy(tmp, ws_hbm.at[pl.ds(g0 + i * _RC, _RC)])
            pltpu.sync_copy(out_hbm.at[c, pl.ds(r0 + i * _RC, _RC)], acc)

            def abody(r, _):
                for q in range(_H // 16):
                    sl = pl.ds(q * 16, 16)
                    acc[r, sl] = acc[r, sl] + fk * tmp[r, sl]
                return _
            lax.fori_loop(0, _RC, abody, 0)
            pltpu.sync_copy(acc, out_hbm.at[c, pl.ds(r0 + i * _RC, _RC)])
        plsc.subcore_barrier()
        return _

    lax.fori_loop(0, _K, hop, 0)


def _sc_prop(h, col, row, w, fw):
    mesh = plsc.VectorSubcoreMesh(core_axis_name="c", subcore_axis_name="s",
                                  num_cores=2, num_subcores=_NT)
    f = pl.kernel(
        _sc_body,
        out_type=[
            jax.ShapeDtypeStruct((2, _NP, _H), jnp.float32),
            jax.ShapeDtypeStruct((2 * _NP, _H), jnp.float32),
        ],
        mesh=mesh,
        compiler_params=pltpu.CompilerParams(use_tc_tiling_on_sc=False),
        scratch_types=[
            pltpu.VMEM_SHARED((_NP, _H), jnp.float32),
            pltpu.VMEM((_NCH, _C), jnp.int32),
            pltpu.VMEM((_NCH, _C), jnp.int32),
            pltpu.VMEM((_NCH, _C), jnp.float32),
            pltpu.VMEM((_C, _H), jnp.float32),
            pltpu.VMEM((_RC, _H), jnp.float32),
            pltpu.VMEM((_RC, _H), jnp.float32),
            pltpu.VMEM((16,), jnp.float32),
        ],
    )
    out, _ws = f(h, col, row, w, fw)
    return out


def _prep_edges(edge_index, edge_weight, level):
    col = edge_index[1].astype(jnp.int32).reshape(_NT, _EPT)
    row = edge_index[0].astype(jnp.int32).reshape(_NT, _EPT)
    w = edge_weight.astype(jnp.float32).reshape(_NT, _EPT)
    pad = ((0, 0), (0, _EPAD - _EPT))
    col = jnp.pad(col, pad).reshape(_NT, _NCH, _C) + level * _NP
    row = jnp.pad(row, pad).reshape(_NT, _NCH, _C)
    w = jnp.pad(w, pad).reshape(_NT, _NCH, _C)
    return col, row, w


def kernel(x, hl1_edge_index, hl1_edge_weight, hl2_edge_index, hl2_edge_weight,
           W_in1, b_in1, fW1, W_in2, b_in2, fW2, W_out, b_out):
    xp = jnp.pad(x, ((0, _NP - _N), (0, 0)))
    h = _proj_in(xp, jnp.stack([W_in1, W_in2]),
                 jnp.stack([b_in1, b_in2])[:, None, :])

    c1, r1, w1 = _prep_edges(hl1_edge_index, hl1_edge_weight, 0)
    c2, r2, w2 = _prep_edges(hl2_edge_index, hl2_edge_weight, 1)
    col = jnp.stack([c1, c2])
    row = jnp.stack([r1, r2])
    w = jnp.stack([w1, w2])
    fw = jnp.stack([
        jnp.pad(fW1, (0, 16 - (_K + 1)), constant_values=-1e30),
        jnp.pad(fW2, (0, 16 - (_K + 1)), constant_values=-1e30),
    ])

    out = _sc_prop(h, col, row, w, fw)[:, :_N]
    return _proj_out(out, W_out, b_out)


# speedup vs baseline: 3.6426x; 1.2063x over previous
import functools

import jax
import jax.numpy as jnp
from jax import lax
from jax.experimental import pallas as pl
from jax.experimental.pallas import tpu as pltpu, tpu_sc as plsc

_N = 10000
_NP = 10240
_E = 320000
_D = 128
_H = 64
_K = 10
_NT = 16
_EPT = _E // _NT
_C = 128
_NCH = 158
_EPAD = _NCH * _C
_RPT = _NP // _NT
_RC = 64


def _proj_in_body(x_ref, w_ref, b_ref, o_ref):
    acc = jnp.dot(x_ref[...], w_ref[0], preferred_element_type=jnp.float32)
    o_ref[...] = acc + b_ref[0]


def _proj_in(x, W, b):
    bn = 1024
    grid = (2, _NP // bn)
    return pl.pallas_call(
        _proj_in_body,
        grid=grid,
        in_specs=[
            pl.BlockSpec((bn, _D), lambda c, i: (i, 0)),
            pl.BlockSpec((1, _D, _H), lambda c, i: (c, 0, 0)),
            pl.BlockSpec((1, 1, _H), lambda c, i: (c, 0, 0)),
        ],
        out_specs=pl.BlockSpec((bn, _H), lambda c, i: (c * (_NP // bn) + i, 0)),
        out_shape=jax.ShapeDtypeStruct((2 * _NP, _H), jnp.float32),
    )(x, W, b)


def _proj_out_body(o_ref, w_ref, b_ref, y_ref):
    z1 = jnp.dot(o_ref[0], w_ref[: _H], preferred_element_type=jnp.float32)
    z2 = jnp.dot(o_ref[1], w_ref[_H:], preferred_element_type=jnp.float32)
    y_ref[...] = z1 + z2 + b_ref[...][None, :]


def _proj_out(o, W_out, b_out):
    bn = 1000
    return pl.pallas_call(
        _proj_out_body,
        grid=(_N // bn,),
        in_specs=[
            pl.BlockSpec((2, bn, _H), lambda i: (0, i, 0)),
            pl.BlockSpec((_H * 2, _H), lambda i: (0, 0)),
            pl.BlockSpec((_H,), lambda i: (0,)),
        ],
        out_specs=pl.BlockSpec((bn, _H), lambda i: (i, 0)),
        out_shape=jax.ShapeDtypeStruct((_N, _H), jnp.float32),
    )(o, W_out, b_out)


def _sc_body(h_hbm, col_hbm, row_hbm, w_hbm, fw_hbm,
             out_hbm, ws_hbm,
             xsp, col_buf, row_buf, w_buf, gbuf0, gbuf1, acc, tmp, fbuf,
             gsem0, gsem1, ssem0, ssem1):
    c = lax.axis_index("c")
    s = lax.axis_index("s")
    r0 = s * _RPT
    g0 = c * _NP + r0

    pltpu.sync_copy(col_hbm.at[c, s], col_buf)
    pltpu.sync_copy(row_hbm.at[c, s], row_buf)
    pltpu.sync_copy(w_hbm.at[c, s], w_buf)

    pltpu.sync_copy(fw_hbm.at[c], fbuf)
    fv = fbuf[...]
    ev = jnp.exp(fv)
    lanes = lax.iota(jnp.int32, 16)
    t = ev
    for sh in (1, 2, 4, 8):
        perm = jnp.bitwise_xor(lanes, sh)
        t = t + t.at[perm].get(mode="promise_in_bounds")
    f = ev / t

    f0 = f[0]
    for i in range(_RPT // _RC):
        pltpu.sync_copy(h_hbm.at[pl.ds(g0 + i * _RC, _RC)], tmp)
        pltpu.sync_copy(tmp, ws_hbm.at[pl.ds(g0 + i * _RC, _RC)])

        def ibody(r, _):
            for q in range(_H // 16):
                sl = pl.ds(q * 16, 16)
                acc[r, sl] = f0 * tmp[r, sl]
            return _
        lax.fori_loop(0, _RC, ibody, 0)
        pltpu.sync_copy(acc, out_hbm.at[c, pl.ds(r0 + i * _RC, _RC)])

    def hop(k, _):
        def zbody(r, _):
            for q in range(_H // 16):
                tmp[r, pl.ds(q * 16, 16)] = jnp.zeros((16,), jnp.float32)
            return _
        lax.fori_loop(0, _RC, zbody, 0)
        for i in range(_RPT // _RC):
            pltpu.sync_copy(tmp, xsp.at[pl.ds(r0 + i * _RC, _RC)])
        plsc.subcore_barrier()

        bufs = (gbuf0, gbuf1)
        gsems = (gsem0, gsem1)
        ssems = (ssem0, ssem1)

        def start_gather(j, b):
            pltpu.async_copy(ws_hbm.at[col_buf.at[j]], bufs[b], gsems[b])

        def wait_gather(b):
            pltpu.make_async_copy(
                ws_hbm.at[col_buf.at[0]], bufs[b], gsems[b]).wait()

        def start_scatter(j, b):
            pltpu.async_copy(bufs[b], xsp.at[row_buf.at[j]], ssems[b],
                             add=True)

        def wait_scatter(b):
            pltpu.make_async_copy(
                bufs[b], xsp.at[row_buf.at[0]], ssems[b]).wait()

        start_gather(0, 0)

        def pair(jo, _):
            for b in range(2):
                j = 2 * jo + b
                bn = 1 - b
                wait_gather(b)

                @pl.when(j >= 1)
                def _w():
                    wait_scatter(bn)

                @pl.when(j + 1 < _NCH)
                def _g():
                    start_gather(j + 1, bn)

                def mbody(g, _, b=b, j=j):
                    wv = w_buf[j, pl.ds(g * 16, 16)]
                    for lane in range(16):
                        w_s = wv[lane]
                        e = g * 16 + lane
                        for q in range(_H // 16):
                            sl = pl.ds(q * 16, 16)
                            bufs[b][e, sl] = bufs[b][e, sl] * w_s
                    return _
                lax.fori_loop(0, _C // 16, mbody, 0)
                start_scatter(j, b)
            return _
        lax.fori_loop(0, _NCH // 2, pair, 0)
        wait_scatter(1)
        plsc.subcore_barrier()

        fk = f.at[jnp.full((16,), k + 1, dtype=jnp.int32)].get(
            mode="promise_in_bounds")
        for i in range(_RPT // _RC):
            pltpu.sync_copy(xsp.at[pl.ds(r0 + i * _RC, _RC)], tmp)
            pltpu.sync_copy(tmp, ws_hbm.at[pl.ds(g0 + i * _RC, _RC)])
            pltpu.sync_copy(out_hbm.at[c, pl.ds(r0 + i * _RC, _RC)], acc)

            def abody(r, _):
                for q in range(_H // 16):
                    sl = pl.ds(q * 16, 16)
                    acc[r, sl] = acc[r, sl] + fk * tmp[r, sl]
                return _
            lax.fori_loop(0, _RC, abody, 0)
            pltpu.sync_copy(acc, out_hbm.at[c, pl.ds(r0 + i * _RC, _RC)])
        plsc.subcore_barrier()
        return _

    lax.fori_loop(0, _K, hop, 0)


def _sc_prop(h, col, row, w, fw):
    mesh = plsc.VectorSubcoreMesh(core_axis_name="c", subcore_axis_name="s",
                                  num_cores=2, num_subcores=_NT)
    f = pl.kernel(
        _sc_body,
        out_type=[
            jax.ShapeDtypeStruct((2, _NP, _H), jnp.float32),
            jax.ShapeDtypeStruct((2 * _NP, _H), jnp.float32),
        ],
        mesh=mesh,
        compiler_params=pltpu.CompilerParams(use_tc_tiling_on_sc=False),
        scratch_types=[
            pltpu.VMEM_SHARED((_NP, _H), jnp.float32),
            pltpu.VMEM((_NCH, _C), jnp.int32),
            pltpu.VMEM((_NCH, _C), jnp.int32),
            pltpu.VMEM((_NCH, _C), jnp.float32),
            pltpu.VMEM((_C, _H), jnp.float32),
            pltpu.VMEM((_C, _H), jnp.float32),
            pltpu.VMEM((_RC, _H), jnp.float32),
            pltpu.VMEM((_RC, _H), jnp.float32),
            pltpu.VMEM((16,), jnp.float32),
            pltpu.SemaphoreType.DMA,
            pltpu.SemaphoreType.DMA,
            pltpu.SemaphoreType.DMA,
            pltpu.SemaphoreType.DMA,
        ],
    )
    out, _ws = f(h, col, row, w, fw)
    return out


def _prep_edges(edge_index, edge_weight, level):
    col = edge_index[1].astype(jnp.int32).reshape(_NT, _EPT)
    row = edge_index[0].astype(jnp.int32).reshape(_NT, _EPT)
    w = edge_weight.astype(jnp.float32).reshape(_NT, _EPT)
    pad = ((0, 0), (0, _EPAD - _EPT))
    col = jnp.pad(col, pad).reshape(_NT, _NCH, _C) + level * _NP
    row = jnp.pad(row, pad).reshape(_NT, _NCH, _C)
    w = jnp.pad(w, pad).reshape(_NT, _NCH, _C)
    return col, row, w


def kernel(x, hl1_edge_index, hl1_edge_weight, hl2_edge_index, hl2_edge_weight,
           W_in1, b_in1, fW1, W_in2, b_in2, fW2, W_out, b_out):
    xp = jnp.pad(x, ((0, _NP - _N), (0, 0)))
    h = _proj_in(xp, jnp.stack([W_in1, W_in2]),
                 jnp.stack([b_in1, b_in2])[:, None, :])

    c1, r1, w1 = _prep_edges(hl1_edge_index, hl1_edge_weight, 0)
    c2, r2, w2 = _prep_edges(hl2_edge_index, hl2_edge_weight, 1)
    col = jnp.stack([c1, c2])
    row = jnp.stack([r1, r2])
    w = jnp.stack([w1, w2])
    fw = jnp.stack([
        jnp.pad(fW1, (0, 16 - (_K + 1)), constant_values=-1e30),
        jnp.pad(fW2, (0, 16 - (_K + 1)), constant_values=-1e30),
    ])

    out = _sc_prop(h, col, row, w, fw)[:, :_N]
    return _proj_out(out, W_out, b_out)


# speedup vs baseline: 3.6592x; 1.0046x over previous
import jax
import jax.numpy as jnp
from jax import lax
from jax.experimental import pallas as pl
from jax.experimental.pallas import tpu as pltpu, tpu_sc as plsc

_N = 10000
_NP = 10240
_E = 320000
_D = 128
_H = 64
_K = 10
_NT = 16
_EPT = _E // _NT
_C = 96
_NCH = 212
_EPAD = _NCH * _C
_RPT = _NP // _NT
_ZC = 64


def _proj_in_body(x_ref, w_ref, b_ref, o_ref):
    acc = jnp.dot(x_ref[...], w_ref[0], preferred_element_type=jnp.float32)
    o_ref[...] = acc + b_ref[0]


def _proj_in(x, W, b):
    bn = 1024
    grid = (2, _NP // bn)
    return pl.pallas_call(
        _proj_in_body,
        grid=grid,
        in_specs=[
            pl.BlockSpec((bn, _D), lambda c, i: (i, 0)),
            pl.BlockSpec((1, _D, _H), lambda c, i: (c, 0, 0)),
            pl.BlockSpec((1, 1, _H), lambda c, i: (c, 0, 0)),
        ],
        out_specs=pl.BlockSpec((bn, _H), lambda c, i: (c * (_NP // bn) + i, 0)),
        out_shape=jax.ShapeDtypeStruct((2 * _NP, _H), jnp.float32),
    )(x, W, b)


def _epi_body(ws_ref, fw_ref, w_ref, b_ref, y_ref):
    fw = fw_ref[...]
    f = jax.nn.softmax(fw, axis=1)
    s1 = jnp.zeros_like(ws_ref[0, 0])
    s2 = jnp.zeros_like(ws_ref[0, 1])
    for k in range(_K + 1):
        s1 = s1 + f[0, k] * ws_ref[k, 0]
        s2 = s2 + f[1, k] * ws_ref[k, 1]
    z1 = jnp.dot(s1, w_ref[: _H], preferred_element_type=jnp.float32)
    z2 = jnp.dot(s2, w_ref[_H:], preferred_element_type=jnp.float32)
    y_ref[...] = z1 + z2 + b_ref[...][None, :]


def _epilogue(ws, fw, W_out, b_out):
    bn = 1000
    return pl.pallas_call(
        _epi_body,
        grid=(_N // bn,),
        in_specs=[
            pl.BlockSpec((_K + 1, 2, bn, _H), lambda i: (0, 0, i, 0)),
            pl.BlockSpec((2, 16), lambda i: (0, 0)),
            pl.BlockSpec((_H * 2, _H), lambda i: (0, 0)),
            pl.BlockSpec((_H,), lambda i: (0,)),
        ],
        out_specs=pl.BlockSpec((bn, _H), lambda i: (i, 0)),
        out_shape=jax.ShapeDtypeStruct((_N, _H), jnp.float32),
    )(ws, fw, W_out, b_out)


def _sc_body(h_hbm, col_hbm, row_hbm, w_hbm, ws_hbm,
             xsp, col_buf, row_buf, w_buf,
             gbuf0, gbuf1, gbuf2, gbuf3, tmp,
             gsem0, gsem1, gsem2, gsem3, ssem0, ssem1, ssem2, ssem3):
    c = lax.axis_index("c")
    s = lax.axis_index("s")
    r0 = s * _RPT
    g0 = c * _NP + r0

    pltpu.sync_copy(col_hbm.at[c, s], col_buf)
    pltpu.sync_copy(row_hbm.at[c, s], row_buf)
    pltpu.sync_copy(w_hbm.at[c, s], w_buf)

    pltpu.sync_copy(h_hbm.at[pl.ds(g0, _RPT)], ws_hbm.at[pl.ds(g0, _RPT)])

    def zb(r, _):
        for q in range(_H // 16):
            tmp[r, pl.ds(q * 16, 16)] = jnp.zeros((16,), jnp.float32)
        return _
    lax.fori_loop(0, _ZC, zb, 0)

    bufs = (gbuf0, gbuf1, gbuf2, gbuf3)
    gsems = (gsem0, gsem1, gsem2, gsem3)
    ssems = (ssem0, ssem1, ssem2, ssem3)

    def start_gather(j, b):
        pltpu.async_copy(ws_hbm.at[col_buf.at[j]], bufs[b], gsems[b])

    def wait_gather(b):
        pltpu.make_async_copy(
            ws_hbm.at[col_buf.at[0]], bufs[b], gsems[b]).wait()

    def start_scatter(j, b):
        pltpu.async_copy(bufs[b], xsp.at[row_buf.at[j]], ssems[b], add=True)

    def wait_scatter(b):
        pltpu.make_async_copy(
            bufs[b], xsp.at[row_buf.at[0]], ssems[b]).wait()

    def hop(k, _):
        for i in range(_RPT // _ZC):
            pltpu.sync_copy(tmp, xsp.at[pl.ds(r0 + i * _ZC, _ZC)])
        plsc.subcore_barrier()

        start_gather(0, 0)
        start_gather(1, 1)

        def ring(jo, _):
            for b in range(4):
                j = 4 * jo + b
                bn = (b + 2) % 4
                wait_gather(b)

                @pl.when(j >= 2)
                def _w():
                    wait_scatter(bn)

                @pl.when(j + 2 < _NCH)
                def _g():
                    start_gather(j + 2, bn)

                def mbody(g, _, b=b, j=j):
                    wv = w_buf[j, pl.ds(g * 16, 16)]
                    for lane in range(16):
                        w_s = wv[lane]
                        e = g * 16 + lane
                        for q in range(_H // 16):
                            sl = pl.ds(q * 16, 16)
                            bufs[b][e, sl] = bufs[b][e, sl] * w_s
                    return _
                lax.fori_loop(0, _C // 16, mbody, 0)
                start_scatter(j, b)
            return _
        lax.fori_loop(0, _NCH // 4, ring, 0)
        wait_scatter(2)
        wait_scatter(3)
        plsc.subcore_barrier()

        dst = (k + 1) * 2 * _NP + g0
        pltpu.sync_copy(xsp.at[pl.ds(r0, _RPT)], ws_hbm.at[pl.ds(dst, _RPT)])

        def adv(j, _):
            for g in range(_C // 16):
                sl = pl.ds(g * 16, 16)
                col_buf[j, sl] = col_buf[j, sl] + (2 * _NP)
            return _
        lax.fori_loop(0, _NCH, adv, 0)
        plsc.subcore_barrier()
        return _

    lax.fori_loop(0, _K, hop, 0)


def _sc_prop(h, col, row, w):
    mesh = plsc.VectorSubcoreMesh(core_axis_name="c", subcore_axis_name="s",
                                  num_cores=2, num_subcores=_NT)
    f = pl.kernel(
        _sc_body,
        out_type=jax.ShapeDtypeStruct(((_K + 1) * 2 * _NP, _H), jnp.float32),
        mesh=mesh,
        compiler_params=pltpu.CompilerParams(use_tc_tiling_on_sc=False),
        scratch_types=[
            pltpu.VMEM_SHARED((_NP, _H), jnp.float32),
            pltpu.VMEM((_NCH, _C), jnp.int32),
            pltpu.VMEM((_NCH, _C), jnp.int32),
            pltpu.VMEM((_NCH, _C), jnp.float32),
            pltpu.VMEM((_C, _H), jnp.float32),
            pltpu.VMEM((_C, _H), jnp.float32),
            pltpu.VMEM((_C, _H), jnp.float32),
            pltpu.VMEM((_C, _H), jnp.float32),
            pltpu.VMEM((_ZC, _H), jnp.float32),
            pltpu.SemaphoreType.DMA,
            pltpu.SemaphoreType.DMA,
            pltpu.SemaphoreType.DMA,
            pltpu.SemaphoreType.DMA,
            pltpu.SemaphoreType.DMA,
            pltpu.SemaphoreType.DMA,
            pltpu.SemaphoreType.DMA,
            pltpu.SemaphoreType.DMA,
        ],
    )
    return f(h, col, row, w)


def _prep_edges(edge_index, edge_weight, level):
    col = edge_index[1].astype(jnp.int32).reshape(_NT, _EPT)
    row = edge_index[0].astype(jnp.int32).reshape(_NT, _EPT)
    w = edge_weight.astype(jnp.float32).reshape(_NT, _EPT)
    pad = ((0, 0), (0, _EPAD - _EPT))
    col = jnp.pad(col, pad).reshape(_NT, _NCH, _C) + level * _NP
    row = jnp.pad(row, pad).reshape(_NT, _NCH, _C)
    w = jnp.pad(w, pad).reshape(_NT, _NCH, _C)
    return col, row, w


def kernel(x, hl1_edge_index, hl1_edge_weight, hl2_edge_index, hl2_edge_weight,
           W_in1, b_in1, fW1, W_in2, b_in2, fW2, W_out, b_out):
    xp = jnp.pad(x, ((0, _NP - _N), (0, 0)))
    h = _proj_in(xp, jnp.stack([W_in1, W_in2]),
                 jnp.stack([b_in1, b_in2])[:, None, :])

    c1, r1, w1 = _prep_edges(hl1_edge_index, hl1_edge_weight, 0)
    c2, r2, w2 = _prep_edges(hl2_edge_index, hl2_edge_weight, 1)
    col = jnp.stack([c1, c2])
    row = jnp.stack([r1, r2])
    w = jnp.stack([w1, w2])

    ws = _sc_prop(h, col, row, w)
    ws = ws.reshape(_K + 1, 2, _NP, _H)

    fw = jnp.stack([
        jnp.pad(fW1, (0, 16 - (_K + 1)), constant_values=-1e30),
        jnp.pad(fW2, (0, 16 - (_K + 1)), constant_values=-1e30),
    ])
    return _epilogue(ws, fw, W_out, b_out)


# speedup vs baseline: 4.5948x; 1.2557x over previous
import jax
import jax.numpy as jnp
from jax import lax
from jax.experimental import pallas as pl
from jax.experimental.pallas import tpu as pltpu, tpu_sc as plsc

_N = 10000
_NP = 10240
_E = 320000
_D = 128
_H = 64
_K = 10
_NT = 16
_EPT = _E // _NT
_C = 64
_NCH = 320
_G = 16
_NGRP = _NCH // _G
_ESLOT = 4 * _G
_EPAD = _NCH * _C
_RPT = _NP // _NT
_ZC = 64


def _proj_in_body(x_ref, w_ref, b_ref, o_ref):
    acc = jnp.dot(x_ref[...], w_ref[0], preferred_element_type=jnp.float32)
    o_ref[...] = acc + b_ref[0]


def _proj_in(x, W, b):
    bn = 1024
    grid = (2, _NP // bn)
    return pl.pallas_call(
        _proj_in_body,
        grid=grid,
        in_specs=[
            pl.BlockSpec((bn, _D), lambda c, i: (i, 0)),
            pl.BlockSpec((1, _D, _H), lambda c, i: (c, 0, 0)),
            pl.BlockSpec((1, 1, _H), lambda c, i: (c, 0, 0)),
        ],
        out_specs=pl.BlockSpec((bn, _H), lambda c, i: (c * (_NP // bn) + i, 0)),
        out_shape=jax.ShapeDtypeStruct((2 * _NP, _H), jnp.float32),
    )(x, W, b)


def _epi_body(ws_ref, fw_ref, w_ref, b_ref, y_ref):
    fw = fw_ref[...]
    f = jax.nn.softmax(fw, axis=1)
    s1 = jnp.zeros_like(ws_ref[0, 0])
    s2 = jnp.zeros_like(ws_ref[0, 1])
    for k in range(_K + 1):
        s1 = s1 + f[0, k] * ws_ref[k, 0]
        s2 = s2 + f[1, k] * ws_ref[k, 1]
    z1 = jnp.dot(s1, w_ref[: _H], preferred_element_type=jnp.float32)
    z2 = jnp.dot(s2, w_ref[_H:], preferred_element_type=jnp.float32)
    y_ref[...] = z1 + z2 + b_ref[...][None, :]


def _epilogue(ws, fw, W_out, b_out):
    bn = 1000
    return pl.pallas_call(
        _epi_body,
        grid=(_N // bn,),
        in_specs=[
            pl.BlockSpec((_K + 1, 2, bn, _H), lambda i: (0, 0, i, 0)),
            pl.BlockSpec((2, 16), lambda i: (0, 0)),
            pl.BlockSpec((_H * 2, _H), lambda i: (0, 0)),
            pl.BlockSpec((_H,), lambda i: (0,)),
        ],
        out_specs=pl.BlockSpec((bn, _H), lambda i: (i, 0)),
        out_shape=jax.ShapeDtypeStruct((_N, _H), jnp.float32),
    )(ws, fw, W_out, b_out)


def _sc_body(h_hbm, col_hbm, row_hbm, w_hbm, ws_hbm,
             xa, xb, col_buf, row_buf, w_buf,
             gbuf0, gbuf1, gbuf2, gbuf3, tmp,
             gsem0, gsem1, gsem2, gsem3, ssem0, ssem1, ssem2, ssem3,
             esem0, esem1, esem2, esem3):
    c = lax.axis_index("c")
    s = lax.axis_index("s")
    r0 = s * _RPT
    g0 = c * _NP + r0
    esems = (esem0, esem1, esem2, esem3)

    def start_edges(g, slot):
        src = pl.ds(g * _G, _G)
        dst = pl.ds(slot * _G, _G)
        pltpu.async_copy(col_hbm.at[c, s, src], col_buf.at[dst], esems[slot])
        pltpu.async_copy(row_hbm.at[c, s, src], row_buf.at[dst], esems[slot])
        pltpu.async_copy(w_hbm.at[c, s, src], w_buf.at[dst], esems[slot])

    def wait_edges(slot):
        src = pl.ds(0, _G)
        dst = pl.ds(slot * _G, _G)
        pltpu.make_async_copy(
            col_hbm.at[c, s, src], col_buf.at[dst], esems[slot]).wait()
        pltpu.make_async_copy(
            row_hbm.at[c, s, src], row_buf.at[dst], esems[slot]).wait()
        pltpu.make_async_copy(
            w_hbm.at[c, s, src], w_buf.at[dst], esems[slot]).wait()

    def edges_dyn(op, sel):
        for i in range(4):
            pl.when(sel == i)(lambda i=i: op(i))

    pltpu.sync_copy(h_hbm.at[pl.ds(g0, _RPT)], ws_hbm.at[pl.ds(g0, _RPT)])
    pltpu.sync_copy(h_hbm.at[pl.ds(g0, _RPT)], xa.at[pl.ds(r0, _RPT)])

    def zb(r, _):
        for q in range(_H // 16):
            tmp[r, pl.ds(q * 16, 16)] = jnp.zeros((16,), jnp.float32)
        return _
    lax.fori_loop(0, _ZC, zb, 0)
    plsc.subcore_barrier()

    bufs = (gbuf0, gbuf1, gbuf2, gbuf3)
    gsems = (gsem0, gsem1, gsem2, gsem3)
    ssems = (ssem0, ssem1, ssem2, ssem3)

    def run_hop(src, dst, k):
        def start_gather(j, b):
            pltpu.async_copy(src.at[col_buf.at[j & (_ESLOT - 1)]],
                             bufs[b], gsems[b])

        def wait_gather(b):
            pltpu.make_async_copy(
                src.at[col_buf.at[0]], bufs[b], gsems[b]).wait()

        def start_scatter(j, b):
            pltpu.async_copy(bufs[b], dst.at[row_buf.at[j & (_ESLOT - 1)]],
                             ssems[b], add=True)

        def wait_scatter(b):
            pltpu.make_async_copy(
                bufs[b], dst.at[row_buf.at[0]], ssems[b]).wait()

        for i in range(_RPT // _ZC):
            pltpu.sync_copy(tmp, dst.at[pl.ds(r0 + i * _ZC, _ZC)])
        plsc.subcore_barrier()

        start_edges(0, 0)
        start_edges(1, 1)
        start_edges(2, 2)
        wait_edges(0)
        start_gather(0, 0)
        start_gather(1, 1)

        def ring(jo, _):
            for b in range(4):
                j = 4 * jo + b
                bn = (b + 2) % 4
                wait_gather(b)

                @pl.when(j >= 2)
                def _w():
                    wait_scatter(bn)

                if b == 1:
                    grp = jo // 4

                    @pl.when((jo % 4 == 0) & (grp + 3 < _NGRP))
                    def _e():
                        edges_dyn(lambda i: start_edges(grp + 3, i),
                                  (grp + 3) % 4)

                    @pl.when((jo % 4 == 3) & (grp + 1 < _NGRP))
                    def _ew():
                        edges_dyn(wait_edges, (grp + 1) % 4)

                @pl.when(j + 2 < _NCH)
                def _g():
                    start_gather(j + 2, bn)

                def mbody(g, _, b=b, j=j):
                    wv = w_buf[j & (_ESLOT - 1), pl.ds(g * 16, 16)]
                    for lane in range(16):
                        w_s = wv[lane]
                        e = g * 16 + lane
                        for q in range(_H // 16):
                            sl = pl.ds(q * 16, 16)
                            bufs[b][e, sl] = bufs[b][e, sl] * w_s
                    return _
                lax.fori_loop(0, _C // 16, mbody, 0)
                start_scatter(j, b)
            return _
        lax.fori_loop(0, _NCH // 4, ring, 0)
        wait_scatter(2)
        wait_scatter(3)
        plsc.subcore_barrier()

        ws0 = (k + 1) * 2 * _NP + g0
        pltpu.sync_copy(dst.at[pl.ds(r0, _RPT)], ws_hbm.at[pl.ds(ws0, _RPT)])

    def hop_pair(kk, _):
        run_hop(xa, xb, 2 * kk)
        run_hop(xb, xa, 2 * kk + 1)
        return _

    lax.fori_loop(0, _K // 2, hop_pair, 0)


def _sc_prop(h, col, row, w):
    mesh = plsc.VectorSubcoreMesh(core_axis_name="c", subcore_axis_name="s",
                                  num_cores=2, num_subcores=_NT)
    f = pl.kernel(
        _sc_body,
        out_type=jax.ShapeDtypeStruct(((_K + 1) * 2 * _NP, _H), jnp.float32),
        mesh=mesh,
        compiler_params=pltpu.CompilerParams(use_tc_tiling_on_sc=False),
        scratch_types=[
            pltpu.VMEM_SHARED((_NP, _H), jnp.float32),
            pltpu.VMEM_SHARED((_NP, _H), jnp.float32),
            pltpu.VMEM((_ESLOT, _C), jnp.int32),
            pltpu.VMEM((_ESLOT, _C), jnp.int32),
            pltpu.VMEM((_ESLOT, _C), jnp.float32),
            pltpu.VMEM((_C, _H), jnp.float32),
            pltpu.VMEM((_C, _H), jnp.float32),
            pltpu.VMEM((_C, _H), jnp.float32),
            pltpu.VMEM((_C, _H), jnp.float32),
            pltpu.VMEM((_ZC, _H), jnp.float32),
            pltpu.SemaphoreType.DMA,
            pltpu.SemaphoreType.DMA,
            pltpu.SemaphoreType.DMA,
            pltpu.SemaphoreType.DMA,
            pltpu.SemaphoreType.DMA,
            pltpu.SemaphoreType.DMA,
            pltpu.SemaphoreType.DMA,
            pltpu.SemaphoreType.DMA,
            pltpu.SemaphoreType.DMA,
            pltpu.SemaphoreType.DMA,
            pltpu.SemaphoreType.DMA,
            pltpu.SemaphoreType.DMA,
        ],
    )
    return f(h, col, row, w)


def _prep_edges(edge_index, edge_weight):
    col = edge_index[1].astype(jnp.int32).reshape(_NT, _EPT)
    row = edge_index[0].astype(jnp.int32).reshape(_NT, _EPT)
    w = edge_weight.astype(jnp.float32).reshape(_NT, _EPT)
    pad = ((0, 0), (0, _EPAD - _EPT))
    col = jnp.pad(col, pad).reshape(_NT, _NCH, _C)
    row = jnp.pad(row, pad).reshape(_NT, _NCH, _C)
    w = jnp.pad(w, pad).reshape(_NT, _NCH, _C)
    return col, row, w


def kernel(x, hl1_edge_index, hl1_edge_weight, hl2_edge_index, hl2_edge_weight,
           W_in1, b_in1, fW1, W_in2, b_in2, fW2, W_out, b_out):
    xp = jnp.pad(x, ((0, _NP - _N), (0, 0)))
    h = _proj_in(xp, jnp.stack([W_in1, W_in2]),
                 jnp.stack([b_in1, b_in2])[:, None, :])

    c1, r1, w1 = _prep_edges(hl1_edge_index, hl1_edge_weight)
    c2, r2, w2 = _prep_edges(hl2_edge_index, hl2_edge_weight)
    col = jnp.stack([c1, c2])
    row = jnp.stack([r1, r2])
    w = jnp.stack([w1, w2])

    ws = _sc_prop(h, col, row, w)
    ws = ws.reshape(_K + 1, 2, _NP, _H)

    fw = jnp.stack([
        jnp.pad(fW1, (0, 16 - (_K + 1)), constant_values=-1e30),
        jnp.pad(fW2, (0, 16 - (_K + 1)), constant_values=-1e30),
    ])
    return _epilogue(ws, fw, W_out, b_out)


# speedup vs baseline: 4.5989x; 1.0009x over previous
import jax
import jax.numpy as jnp
from jax import lax
from jax.experimental import pallas as pl
from jax.experimental.pallas import tpu as pltpu, tpu_sc as plsc

_N = 10000
_NP = 10240
_E = 320000
_D = 128
_H = 64
_K = 10
_NT = 16
_EPT = _E // _NT
_C = 64
_NCH = 320
_G = 16
_NGRP = _NCH // _G
_ESLOT = 4 * _G
_EPAD = _NCH * _C
_RPT = _NP // _NT
_ZC = 32
_NB = 8
_LA = 5


def _proj_in_body(x_ref, w_ref, b_ref, o_ref):
    acc = jnp.dot(x_ref[...], w_ref[0], preferred_element_type=jnp.float32)
    o_ref[...] = acc + b_ref[0]


def _proj_in(x, W, b):
    bn = 1024
    grid = (2, _NP // bn)
    return pl.pallas_call(
        _proj_in_body,
        grid=grid,
        in_specs=[
            pl.BlockSpec((bn, _D), lambda c, i: (i, 0)),
            pl.BlockSpec((1, _D, _H), lambda c, i: (c, 0, 0)),
            pl.BlockSpec((1, 1, _H), lambda c, i: (c, 0, 0)),
        ],
        out_specs=pl.BlockSpec((bn, _H), lambda c, i: (c * (_NP // bn) + i, 0)),
        out_shape=jax.ShapeDtypeStruct((2 * _NP, _H), jnp.float32),
    )(x, W, b)


def _epi_body(ws_ref, fw_ref, w_ref, b_ref, y_ref):
    fw = fw_ref[...]
    f = jax.nn.softmax(fw, axis=1)
    s1 = jnp.zeros_like(ws_ref[0, 0])
    s2 = jnp.zeros_like(ws_ref[0, 1])
    for k in range(_K + 1):
        s1 = s1 + f[0, k] * ws_ref[k, 0]
        s2 = s2 + f[1, k] * ws_ref[k, 1]
    z1 = jnp.dot(s1, w_ref[: _H], preferred_element_type=jnp.float32)
    z2 = jnp.dot(s2, w_ref[_H:], preferred_element_type=jnp.float32)
    y_ref[...] = z1 + z2 + b_ref[...][None, :]


def _epilogue(ws, fw, W_out, b_out):
    bn = 1000
    return pl.pallas_call(
        _epi_body,
        grid=(_N // bn,),
        in_specs=[
            pl.BlockSpec((_K + 1, 2, bn, _H), lambda i: (0, 0, i, 0)),
            pl.BlockSpec((2, 16), lambda i: (0, 0)),
            pl.BlockSpec((_H * 2, _H), lambda i: (0, 0)),
            pl.BlockSpec((_H,), lambda i: (0,)),
        ],
        out_specs=pl.BlockSpec((bn, _H), lambda i: (i, 0)),
        out_shape=jax.ShapeDtypeStruct((_N, _H), jnp.float32),
    )(ws, fw, W_out, b_out)


def _sc_body(h_hbm, col_hbm, row_hbm, w_hbm, ws_hbm,
             xa, xb, col_buf, row_buf, w_buf,
             gbuf0, gbuf1, gbuf2, gbuf3, gbuf4, gbuf5, gbuf6, gbuf7, tmp,
             gsem0, gsem1, gsem2, gsem3, gsem4, gsem5, gsem6, gsem7,
             ssem0, ssem1, ssem2, ssem3,
             esem0, esem1, esem2, esem3):
    c = lax.axis_index("c")
    s = lax.axis_index("s")
    r0 = s * _RPT
    g0 = c * _NP + r0
    esems = (esem0, esem1, esem2, esem3)

    def start_edges(g, slot):
        src = pl.ds(g * _G, _G)
        dst = pl.ds(slot * _G, _G)
        pltpu.async_copy(col_hbm.at[c, s, src], col_buf.at[dst], esems[slot])
        pltpu.async_copy(row_hbm.at[c, s, src], row_buf.at[dst], esems[slot])
        pltpu.async_copy(w_hbm.at[c, s, src], w_buf.at[dst], esems[slot])

    def wait_edges(slot):
        src = pl.ds(0, _G)
        dst = pl.ds(slot * _G, _G)
        pltpu.make_async_copy(
            col_hbm.at[c, s, src], col_buf.at[dst], esems[slot]).wait()
        pltpu.make_async_copy(
            row_hbm.at[c, s, src], row_buf.at[dst], esems[slot]).wait()
        pltpu.make_async_copy(
            w_hbm.at[c, s, src], w_buf.at[dst], esems[slot]).wait()

    def edges_dyn(op, sel):
        for i in range(4):
            pl.when(sel == i)(lambda i=i: op(i))

    pltpu.sync_copy(h_hbm.at[pl.ds(g0, _RPT)], ws_hbm.at[pl.ds(g0, _RPT)])
    pltpu.sync_copy(h_hbm.at[pl.ds(g0, _RPT)], xa.at[pl.ds(r0, _RPT)])

    def zb(r, _):
        for q in range(_H // 16):
            tmp[r, pl.ds(q * 16, 16)] = jnp.zeros((16,), jnp.float32)
        return _
    lax.fori_loop(0, _ZC, zb, 0)
    plsc.subcore_barrier()

    bufs = (gbuf0, gbuf1, gbuf2, gbuf3, gbuf4, gbuf5, gbuf6, gbuf7)
    gsems = (gsem0, gsem1, gsem2, gsem3, gsem4, gsem5, gsem6, gsem7)
    ssems = (ssem0, ssem1, ssem2, ssem3)

    def run_hop(src, dst, k):
        def start_gather(j, b):
            pltpu.async_copy(src.at[col_buf.at[j & (_ESLOT - 1)]],
                             bufs[b], gsems[b])

        def wait_gather(b):
            pltpu.make_async_copy(
                src.at[col_buf.at[0]], bufs[b], gsems[b]).wait()

        def start_scatter(j, b):
            pltpu.async_copy(bufs[b], dst.at[row_buf.at[j & (_ESLOT - 1)]],
                             ssems[b % 4], add=True)

        def wait_scatter(b):
            pltpu.make_async_copy(
                bufs[b], dst.at[row_buf.at[0]], ssems[b % 4]).wait()

        for i in range(_RPT // _ZC):
            pltpu.sync_copy(tmp, dst.at[pl.ds(r0 + i * _ZC, _ZC)])
        plsc.subcore_barrier()

        start_edges(0, 0)
        start_edges(1, 1)
        start_edges(2, 2)
        wait_edges(0)
        for j0 in range(_LA):
            start_gather(j0, j0)

        def ring(jo, _):
            for b in range(_NB):
                j = _NB * jo + b
                bn = (b + _LA) % _NB
                wait_gather(b)

                @pl.when(j >= 3)
                def _w():
                    wait_scatter(bn)

                if b % 8 == 2:
                    jm = j & 15
                    grp = j // 16

                    @pl.when((jm == 2) & (grp + 3 < _NGRP))
                    def _e():
                        edges_dyn(lambda i: start_edges(grp + 3, i),
                                  (grp + 3) % 4)

                    @pl.when((jm == 10) & (grp + 1 < _NGRP))
                    def _ew():
                        edges_dyn(wait_edges, (grp + 1) % 4)

                @pl.when(j + _LA < _NCH)
                def _g():
                    start_gather(j + _LA, bn)

                def mbody(g, _, b=b, j=j):
                    wv = w_buf[j & (_ESLOT - 1), pl.ds(g * 16, 16)]
                    for lane in range(16):
                        w_s = wv[lane]
                        e = g * 16 + lane
                        for q in range(_H // 16):
                            sl = pl.ds(q * 16, 16)
                            bufs[b][e, sl] = bufs[b][e, sl] * w_s
                    return _
                lax.fori_loop(0, _C // 16, mbody, 0)
                start_scatter(j, b)
            return _
        lax.fori_loop(0, _NCH // _NB, ring, 0)
        for jt in range(_NCH - 3, _NCH):
            wait_scatter(jt % _NB)
        plsc.subcore_barrier()

        ws0 = (k + 1) * 2 * _NP + g0
        pltpu.sync_copy(dst.at[pl.ds(r0, _RPT)], ws_hbm.at[pl.ds(ws0, _RPT)])

    def hop_pair(kk, _):
        run_hop(xa, xb, 2 * kk)
        run_hop(xb, xa, 2 * kk + 1)
        return _

    lax.fori_loop(0, _K // 2, hop_pair, 0)


def _sc_prop(h, col, row, w):
    mesh = plsc.VectorSubcoreMesh(core_axis_name="c", subcore_axis_name="s",
                                  num_cores=2, num_subcores=_NT)
    f = pl.kernel(
        _sc_body,
        out_type=jax.ShapeDtypeStruct(((_K + 1) * 2 * _NP, _H), jnp.float32),
        mesh=mesh,
        compiler_params=pltpu.CompilerParams(use_tc_tiling_on_sc=False),
        scratch_types=[
            pltpu.VMEM_SHARED((_NP, _H), jnp.float32),
            pltpu.VMEM_SHARED((_NP, _H), jnp.float32),
            pltpu.VMEM((_ESLOT, _C), jnp.int32),
            pltpu.VMEM((_ESLOT, _C), jnp.int32),
            pltpu.VMEM((_ESLOT, _C), jnp.float32),
            pltpu.VMEM((_C, _H), jnp.float32),
            pltpu.VMEM((_C, _H), jnp.float32),
            pltpu.VMEM((_C, _H), jnp.float32),
            pltpu.VMEM((_C, _H), jnp.float32),
            pltpu.VMEM((_C, _H), jnp.float32),
            pltpu.VMEM((_C, _H), jnp.float32),
            pltpu.VMEM((_C, _H), jnp.float32),
            pltpu.VMEM((_C, _H), jnp.float32),
            pltpu.VMEM((_ZC, _H), jnp.float32),
            pltpu.SemaphoreType.DMA,
            pltpu.SemaphoreType.DMA,
            pltpu.SemaphoreType.DMA,
            pltpu.SemaphoreType.DMA,
            pltpu.SemaphoreType.DMA,
            pltpu.SemaphoreType.DMA,
            pltpu.SemaphoreType.DMA,
            pltpu.SemaphoreType.DMA,
            pltpu.SemaphoreType.DMA,
            pltpu.SemaphoreType.DMA,
            pltpu.SemaphoreType.DMA,
            pltpu.SemaphoreType.DMA,
            pltpu.SemaphoreType.DMA,
            pltpu.SemaphoreType.DMA,
            pltpu.SemaphoreType.DMA,
            pltpu.SemaphoreType.DMA,
        ],
    )
    return f(h, col, row, w)


def _prep_edges(edge_index, edge_weight):
    col = edge_index[1].astype(jnp.int32).reshape(_NT, _EPT)
    row = edge_index[0].astype(jnp.int32).reshape(_NT, _EPT)
    w = edge_weight.astype(jnp.float32).reshape(_NT, _EPT)
    pad = ((0, 0), (0, _EPAD - _EPT))
    col = jnp.pad(col, pad).reshape(_NT, _NCH, _C)
    row = jnp.pad(row, pad).reshape(_NT, _NCH, _C)
    w = jnp.pad(w, pad).reshape(_NT, _NCH, _C)
    return col, row, w


def kernel(x, hl1_edge_index, hl1_edge_weight, hl2_edge_index, hl2_edge_weight,
           W_in1, b_in1, fW1, W_in2, b_in2, fW2, W_out, b_out):
    xp = jnp.pad(x, ((0, _NP - _N), (0, 0)))
    h = _proj_in(xp, jnp.stack([W_in1, W_in2]),
                 jnp.stack([b_in1, b_in2])[:, None, :])

    c1, r1, w1 = _prep_edges(hl1_edge_index, hl1_edge_weight)
    c2, r2, w2 = _prep_edges(hl2_edge_index, hl2_edge_weight)
    col = jnp.stack([c1, c2])
    row = jnp.stack([r1, r2])
    w = jnp.stack([w1, w2])

    ws = _sc_prop(h, col, row, w)
    ws = ws.reshape(_K + 1, 2, _NP, _H)

    fw = jnp.stack([
        jnp.pad(fW1, (0, 16 - (_K + 1)), constant_values=-1e30),
        jnp.pad(fW2, (0, 16 - (_K + 1)), constant_values=-1e30),
    ])
    return _epilogue(ws, fw, W_out, b_out)


# speedup vs baseline: 4.6401x; 1.0089x over previous
import jax
import jax.numpy as jnp
from jax import lax
from jax.experimental import pallas as pl
from jax.experimental.pallas import tpu as pltpu, tpu_sc as plsc

_N = 10000
_NP = 10240
_E = 320000
_D = 128
_H = 64
_K = 10
_NT = 16
_EPT = _E // _NT
_C = 64
_NCH = 320
_G = 16
_NGRP = _NCH // _G
_ESLOT = 4 * _G
_EPAD = _NCH * _C
_RPT = _NP // _NT
_ZC = 32
_NB = 8
_LA = 5


def _proj_in_body(x_ref, w_ref, b_ref, o_ref):
    acc = jnp.dot(x_ref[...], w_ref[0], preferred_element_type=jnp.float32)
    o_ref[...] = acc + b_ref[0]


def _proj_in(x, W, b):
    bn = 1024
    grid = (2, _NP // bn)
    return pl.pallas_call(
        _proj_in_body,
        grid=grid,
        in_specs=[
            pl.BlockSpec((bn, _D), lambda c, i: (i, 0)),
            pl.BlockSpec((1, _D, _H), lambda c, i: (c, 0, 0)),
            pl.BlockSpec((1, 1, _H), lambda c, i: (c, 0, 0)),
        ],
        out_specs=pl.BlockSpec((bn, _H), lambda c, i: (c * (_NP // bn) + i, 0)),
        out_shape=jax.ShapeDtypeStruct((2 * _NP, _H), jnp.float32),
    )(x, W, b)


def _epi_body(ws_ref, fw_ref, w_ref, b_ref, y_ref):
    fw = fw_ref[...]
    f = jax.nn.softmax(fw, axis=1)
    s1 = jnp.zeros_like(ws_ref[0, 0])
    s2 = jnp.zeros_like(ws_ref[0, 1])
    for k in range(_K + 1):
        s1 = s1 + f[0, k] * ws_ref[k, 0]
        s2 = s2 + f[1, k] * ws_ref[k, 1]
    z1 = jnp.dot(s1, w_ref[: _H], preferred_element_type=jnp.float32)
    z2 = jnp.dot(s2, w_ref[_H:], preferred_element_type=jnp.float32)
    y_ref[...] = z1 + z2 + b_ref[...][None, :]


def _epilogue(ws, fw, W_out, b_out):
    bn = 1000
    return pl.pallas_call(
        _epi_body,
        grid=(_N // bn,),
        in_specs=[
            pl.BlockSpec((_K + 1, 2, bn, _H), lambda i: (0, 0, i, 0)),
            pl.BlockSpec((2, 16), lambda i: (0, 0)),
            pl.BlockSpec((_H * 2, _H), lambda i: (0, 0)),
            pl.BlockSpec((_H,), lambda i: (0,)),
        ],
        out_specs=pl.BlockSpec((bn, _H), lambda i: (i, 0)),
        out_shape=jax.ShapeDtypeStruct((_N, _H), jnp.float32),
    )(ws, fw, W_out, b_out)


def _sc_body(h_hbm, col_hbm, row_hbm, w_hbm, ws_hbm,
             xa, xb, col_buf, row_buf, w_buf,
             gbuf0, gbuf1, gbuf2, gbuf3, gbuf4, gbuf5, gbuf6, gbuf7, tmp,
             gsem0, gsem1, gsem2, gsem3, gsem4, gsem5, gsem6, gsem7,
             ssem0, ssem1, ssem2, ssem3,
             esem0, esem1, esem2, esem3):
    c = lax.axis_index("c")
    s = lax.axis_index("s")
    r0 = s * _RPT
    g0 = c * _NP + r0
    esems = (esem0, esem1, esem2, esem3)

    def start_edges(g, slot):
        src = pl.ds(g * _G, _G)
        dst = pl.ds(slot * _G, _G)
        pltpu.async_copy(col_hbm.at[c, s, src], col_buf.at[dst], esems[slot])
        pltpu.async_copy(row_hbm.at[c, s, src], row_buf.at[dst], esems[slot])
        pltpu.async_copy(w_hbm.at[c, s, src], w_buf.at[dst], esems[slot])

    def wait_edges(slot):
        src = pl.ds(0, _G)
        dst = pl.ds(slot * _G, _G)
        pltpu.make_async_copy(
            col_hbm.at[c, s, src], col_buf.at[dst], esems[slot]).wait()
        pltpu.make_async_copy(
            row_hbm.at[c, s, src], row_buf.at[dst], esems[slot]).wait()
        pltpu.make_async_copy(
            w_hbm.at[c, s, src], w_buf.at[dst], esems[slot]).wait()

    def edges_dyn(op, sel):
        for i in range(4):
            pl.when(sel == i)(lambda i=i: op(i))

    pltpu.sync_copy(h_hbm.at[pl.ds(g0, _RPT)], ws_hbm.at[pl.ds(g0, _RPT)])
    pltpu.sync_copy(h_hbm.at[pl.ds(g0, _RPT)], xa.at[pl.ds(r0, _RPT)])

    def zb(r, _):
        for q in range(_H // 16):
            tmp[r, pl.ds(q * 16, 16)] = jnp.zeros((16,), jnp.float32)
        return _
    lax.fori_loop(0, _ZC, zb, 0)
    plsc.subcore_barrier()

    bufs = (gbuf0, gbuf1, gbuf2, gbuf3, gbuf4, gbuf5, gbuf6, gbuf7)
    gsems = (gsem0, gsem1, gsem2, gsem3, gsem4, gsem5, gsem6, gsem7)
    ssems = (ssem0, ssem1, ssem2, ssem3)

    def run_hop(src, dst, k):
        _DMA = False

        def start_gather(j, b):
            if _DMA:
                pltpu.async_copy(src.at[col_buf.at[j & (_ESLOT - 1)]],
                                 bufs[b], gsems[b])

        def wait_gather(b):
            if _DMA:
                pltpu.make_async_copy(
                    src.at[col_buf.at[0]], bufs[b], gsems[b]).wait()

        def start_scatter(j, b):
            if _DMA:
                pltpu.async_copy(bufs[b],
                                 dst.at[row_buf.at[j & (_ESLOT - 1)]],
                                 ssems[b % 4], add=True)

        def wait_scatter(b):
            if _DMA:
                pltpu.make_async_copy(
                    bufs[b], dst.at[row_buf.at[0]], ssems[b % 4]).wait()

        for i in range(_RPT // _ZC):
            pltpu.sync_copy(tmp, dst.at[pl.ds(r0 + i * _ZC, _ZC)])
        plsc.subcore_barrier()

        start_edges(0, 0)
        start_edges(1, 1)
        start_edges(2, 2)
        wait_edges(0)
        for j0 in range(_LA):
            start_gather(j0, j0)

        def ring(jo, _):
            for b in range(_NB):
                j = _NB * jo + b
                bn = (b + _LA) % _NB
                wait_gather(b)

                @pl.when(j >= 3)
                def _w():
                    wait_scatter(bn)

                if b % 8 == 2:
                    jm = j & 15
                    grp = j // 16

                    @pl.when((jm == 2) & (grp + 3 < _NGRP))
                    def _e():
                        edges_dyn(lambda i: start_edges(grp + 3, i),
                                  (grp + 3) % 4)

                    @pl.when((jm == 10) & (grp + 1 < _NGRP))
                    def _ew():
                        edges_dyn(wait_edges, (grp + 1) % 4)

                @pl.when(j + _LA < _NCH)
                def _g():
                    start_gather(j + _LA, bn)

                def mbody(g, _, b=b, j=j):
                    wv = w_buf[j & (_ESLOT - 1), pl.ds(g * 16, 16)]
                    for lane in range(16):
                        w_s = wv[lane]
                        e = g * 16 + lane
                        for q in range(_H // 16):
                            sl = pl.ds(q * 16, 16)
                            bufs[b][e, sl] = bufs[b][e, sl] * w_s
                    return _
                lax.fori_loop(0, _C // 16, mbody, 0)
                start_scatter(j, b)
            return _
        lax.fori_loop(0, _NCH // _NB, ring, 0)
        for jt in range(_NCH - 3, _NCH):
            wait_scatter(jt % _NB)
        plsc.subcore_barrier()

        ws0 = (k + 1) * 2 * _NP + g0
        pltpu.sync_copy(dst.at[pl.ds(r0, _RPT)], ws_hbm.at[pl.ds(ws0, _RPT)])

    def hop_pair(kk, _):
        run_hop(xa, xb, 2 * kk)
        run_hop(xb, xa, 2 * kk + 1)
        return _

    lax.fori_loop(0, _K // 2, hop_pair, 0)


def _sc_prop(h, col, row, w):
    mesh = plsc.VectorSubcoreMesh(core_axis_name="c", subcore_axis_name="s",
                                  num_cores=2, num_subcores=_NT)
    f = pl.kernel(
        _sc_body,
        out_type=jax.ShapeDtypeStruct(((_K + 1) * 2 * _NP, _H), jnp.float32),
        mesh=mesh,
        compiler_params=pltpu.CompilerParams(use_tc_tiling_on_sc=False),
        scratch_types=[
            pltpu.VMEM_SHARED((_NP, _H), jnp.float32),
            pltpu.VMEM_SHARED((_NP, _H), jnp.float32),
            pltpu.VMEM((_ESLOT, _C), jnp.int32),
            pltpu.VMEM((_ESLOT, _C), jnp.int32),
            pltpu.VMEM((_ESLOT, _C), jnp.float32),
            pltpu.VMEM((_C, _H), jnp.float32),
            pltpu.VMEM((_C, _H), jnp.float32),
            pltpu.VMEM((_C, _H), jnp.float32),
            pltpu.VMEM((_C, _H), jnp.float32),
            pltpu.VMEM((_C, _H), jnp.float32),
            pltpu.VMEM((_C, _H), jnp.float32),
            pltpu.VMEM((_C, _H), jnp.float32),
            pltpu.VMEM((_C, _H), jnp.float32),
            pltpu.VMEM((_ZC, _H), jnp.float32),
            pltpu.SemaphoreType.DMA,
            pltpu.SemaphoreType.DMA,
            pltpu.SemaphoreType.DMA,
            pltpu.SemaphoreType.DMA,
            pltpu.SemaphoreType.DMA,
            pltpu.SemaphoreType.DMA,
            pltpu.SemaphoreType.DMA,
            pltpu.SemaphoreType.DMA,
            pltpu.SemaphoreType.DMA,
            pltpu.SemaphoreType.DMA,
            pltpu.SemaphoreType.DMA,
            pltpu.SemaphoreType.DMA,
            pltpu.SemaphoreType.DMA,
            pltpu.SemaphoreType.DMA,
            pltpu.SemaphoreType.DMA,
            pltpu.SemaphoreType.DMA,
        ],
    )
    return f(h, col, row, w)


def _prep_edges(edge_index, edge_weight):
    col = edge_index[1].astype(jnp.int32).reshape(_NT, _EPT)
    row = edge_index[0].astype(jnp.int32).reshape(_NT, _EPT)
    w = edge_weight.astype(jnp.float32).reshape(_NT, _EPT)
    pad = ((0, 0), (0, _EPAD - _EPT))
    col = jnp.pad(col, pad).reshape(_NT, _NCH, _C)
    row = jnp.pad(row, pad).reshape(_NT, _NCH, _C)
    w = jnp.pad(w, pad).reshape(_NT, _NCH, _C)
    return col, row, w


def kernel(x, hl1_edge_index, hl1_edge_weight, hl2_edge_index, hl2_edge_weight,
           W_in1, b_in1, fW1, W_in2, b_in2, fW2, W_out, b_out):
    xp = jnp.pad(x, ((0, _NP - _N), (0, 0)))
    h = _proj_in(xp, jnp.stack([W_in1, W_in2]),
                 jnp.stack([b_in1, b_in2])[:, None, :])

    c1, r1, w1 = _prep_edges(hl1_edge_index, hl1_edge_weight)
    c2, r2, w2 = _prep_edges(hl2_edge_index, hl2_edge_weight)
    col = jnp.stack([c1, c2])
    row = jnp.stack([r1, r2])
    w = jnp.stack([w1, w2])

    ws = _sc_prop(h, col, row, w)
    ws = ws.reshape(_K + 1, 2, _NP, _H)

    fw = jnp.stack([
        jnp.pad(fW1, (0, 16 - (_K + 1)), constant_values=-1e30),
        jnp.pad(fW2, (0, 16 - (_K + 1)), constant_values=-1e30),
    ])
    return _epilogue(ws, fw, W_out, b_out)


# speedup vs baseline: 8.7428x; 1.8842x over previous
import jax
import jax.numpy as jnp
from jax import lax
from jax.experimental import pallas as pl
from jax.experimental.pallas import tpu as pltpu, tpu_sc as plsc

_N = 10000
_NP = 10240
_E = 320000
_D = 128
_H = 64
_K = 10
_NT = 16
_EPT = _E // _NT
_C = 64
_NCH = 320
_G = 16
_NGRP = _NCH // _G
_ESLOT = 4 * _G
_EPAD = _NCH * _C
_RPT = _NP // _NT
_ZC = 32
_NB = 8
_LA = 5


def _proj_in_body(x_ref, w_ref, b_ref, o_ref):
    acc = jnp.dot(x_ref[...], w_ref[0], preferred_element_type=jnp.float32)
    o_ref[...] = acc + b_ref[0]


def _proj_in(x, W, b):
    bn = 1024
    grid = (2, _NP // bn)
    return pl.pallas_call(
        _proj_in_body,
        grid=grid,
        in_specs=[
            pl.BlockSpec((bn, _D), lambda c, i: (i, 0)),
            pl.BlockSpec((1, _D, _H), lambda c, i: (c, 0, 0)),
            pl.BlockSpec((1, 1, _H), lambda c, i: (c, 0, 0)),
        ],
        out_specs=pl.BlockSpec((bn, _H), lambda c, i: (c * (_NP // bn) + i, 0)),
        out_shape=jax.ShapeDtypeStruct((2 * _NP, _H), jnp.float32),
    )(x, W, b)


def _epi_body(ws_ref, fw_ref, w_ref, b_ref, y_ref):
    fw = fw_ref[...]
    f = jax.nn.softmax(fw, axis=1)
    s1 = jnp.zeros_like(ws_ref[0, 0])
    s2 = jnp.zeros_like(ws_ref[0, 1])
    for k in range(_K + 1):
        s1 = s1 + f[0, k] * ws_ref[k, 0]
        s2 = s2 + f[1, k] * ws_ref[k, 1]
    z1 = jnp.dot(s1, w_ref[: _H], preferred_element_type=jnp.float32)
    z2 = jnp.dot(s2, w_ref[_H:], preferred_element_type=jnp.float32)
    y_ref[...] = z1 + z2 + b_ref[...][None, :]


def _epilogue(ws, fw, W_out, b_out):
    bn = 1000
    return pl.pallas_call(
        _epi_body,
        grid=(_N // bn,),
        in_specs=[
            pl.BlockSpec((_K + 1, 2, bn, _H), lambda i: (0, 0, i, 0)),
            pl.BlockSpec((2, 16), lambda i: (0, 0)),
            pl.BlockSpec((_H * 2, _H), lambda i: (0, 0)),
            pl.BlockSpec((_H,), lambda i: (0,)),
        ],
        out_specs=pl.BlockSpec((bn, _H), lambda i: (i, 0)),
        out_shape=jax.ShapeDtypeStruct((_N, _H), jnp.float32),
    )(ws, fw, W_out, b_out)


def _sc_body(h_hbm, col_hbm, row_hbm, w_hbm, ws_hbm,
             xa, xb, col_buf, row_buf, w_buf,
             gbuf0, gbuf1, gbuf2, gbuf3, gbuf4, gbuf5, gbuf6, gbuf7, tmp,
             gsem0, gsem1, gsem2, gsem3, gsem4, gsem5, gsem6, gsem7,
             ssem0, ssem1, ssem2, ssem3,
             esem0, esem1, esem2, esem3):
    c = lax.axis_index("c")
    s = lax.axis_index("s")
    r0 = s * _RPT
    g0 = c * _NP + r0
    esems = (esem0, esem1, esem2, esem3)

    def start_edges(g, slot):
        src = pl.ds(g * _G, _G)
        dst = pl.ds(slot * _G, _G)
        pltpu.async_copy(col_hbm.at[c, s, src], col_buf.at[dst], esems[slot])
        pltpu.async_copy(row_hbm.at[c, s, src], row_buf.at[dst], esems[slot])
        pltpu.async_copy(w_hbm.at[c, s, src], w_buf.at[dst], esems[slot])

    def wait_edges(slot):
        src = pl.ds(0, _G)
        dst = pl.ds(slot * _G, _G)
        pltpu.make_async_copy(
            col_hbm.at[c, s, src], col_buf.at[dst], esems[slot]).wait()
        pltpu.make_async_copy(
            row_hbm.at[c, s, src], row_buf.at[dst], esems[slot]).wait()
        pltpu.make_async_copy(
            w_hbm.at[c, s, src], w_buf.at[dst], esems[slot]).wait()

    def edges_dyn(op, sel):
        for i in range(4):
            pl.when(sel == i)(lambda i=i: op(i))

    pltpu.sync_copy(h_hbm.at[pl.ds(g0, _RPT)], ws_hbm.at[pl.ds(g0, _RPT)])
    pltpu.sync_copy(h_hbm.at[pl.ds(g0, _RPT)], xa.at[pl.ds(r0, _RPT)])

    def zb(r, _):
        for q in range(_H // 16):
            tmp[r, pl.ds(q * 16, 16)] = jnp.zeros((16,), jnp.float32)
        return _
    lax.fori_loop(0, _ZC, zb, 0)
    plsc.subcore_barrier()

    bufs = (gbuf0, gbuf1, gbuf2, gbuf3, gbuf4, gbuf5, gbuf6, gbuf7)
    gsems = (gsem0, gsem1, gsem2, gsem3, gsem4, gsem5, gsem6, gsem7)
    ssems = (ssem0, ssem1, ssem2, ssem3)

    def run_hop(src, dst, k):
        def start_gather(j, b):
            pltpu.async_copy(src.at[col_buf.at[j & (_ESLOT - 1)]],
                             bufs[b], gsems[b])

        def wait_gather(b):
            pltpu.make_async_copy(
                src.at[col_buf.at[0]], bufs[b], gsems[b]).wait()

        def start_scatter(j, b):
            pltpu.async_copy(bufs[b],
                             dst.at[row_buf.at[j & (_ESLOT - 1)]],
                             ssems[b % 4], add=True)

        def wait_scatter(b):
            pltpu.make_async_copy(
                bufs[b], dst.at[row_buf.at[0]], ssems[b % 4]).wait()

        for i in range(_RPT // _ZC):
            pltpu.sync_copy(tmp, dst.at[pl.ds(r0 + i * _ZC, _ZC)])
        plsc.subcore_barrier()

        start_edges(0, 0)
        start_edges(1, 1)
        start_edges(2, 2)
        wait_edges(0)
        for j0 in range(_LA):
            start_gather(j0, j0)

        def ring(jo, _):
            for b in range(_NB):
                j = _NB * jo + b
                bn = (b + _LA) % _NB
                wait_gather(b)

                @pl.when(j >= 3)
                def _w():
                    wait_scatter(bn)

                if b % 8 == 2:
                    jm = j & 15
                    grp = j // 16

                    @pl.when((jm == 2) & (grp + 3 < _NGRP))
                    def _e():
                        edges_dyn(lambda i: start_edges(grp + 3, i),
                                  (grp + 3) % 4)

                    @pl.when((jm == 10) & (grp + 1 < _NGRP))
                    def _ew():
                        edges_dyn(wait_edges, (grp + 1) % 4)

                @pl.when(j + _LA < _NCH)
                def _g():
                    start_gather(j + _LA, bn)

                for g in range(_C // 16):
                    wv = w_buf[j & (_ESLOT - 1), pl.ds(g * 16, 16)]
                    for lane in range(16):
                        w_s = wv[lane]
                        e = g * 16 + lane
                        for q in range(_H // 16):
                            sl = pl.ds(q * 16, 16)
                            bufs[b][e, sl] = bufs[b][e, sl] * w_s
                start_scatter(j, b)
            return _
        lax.fori_loop(0, _NCH // _NB, ring, 0)
        for jt in range(_NCH - 3, _NCH):
            wait_scatter(jt % _NB)
        plsc.subcore_barrier()

        ws0 = (k + 1) * 2 * _NP + g0
        pltpu.sync_copy(dst.at[pl.ds(r0, _RPT)], ws_hbm.at[pl.ds(ws0, _RPT)])

    def hop_pair(kk, _):
        run_hop(xa, xb, 2 * kk)
        run_hop(xb, xa, 2 * kk + 1)
        return _

    lax.fori_loop(0, _K // 2, hop_pair, 0)


def _sc_prop(h, col, row, w):
    mesh = plsc.VectorSubcoreMesh(core_axis_name="c", subcore_axis_name="s",
                                  num_cores=2, num_subcores=_NT)
    f = pl.kernel(
        _sc_body,
        out_type=jax.ShapeDtypeStruct(((_K + 1) * 2 * _NP, _H), jnp.float32),
        mesh=mesh,
        compiler_params=pltpu.CompilerParams(use_tc_tiling_on_sc=False),
        scratch_types=[
            pltpu.VMEM_SHARED((_NP, _H), jnp.float32),
            pltpu.VMEM_SHARED((_NP, _H), jnp.float32),
            pltpu.VMEM((_ESLOT, _C), jnp.int32),
            pltpu.VMEM((_ESLOT, _C), jnp.int32),
            pltpu.VMEM((_ESLOT, _C), jnp.float32),
            pltpu.VMEM((_C, _H), jnp.float32),
            pltpu.VMEM((_C, _H), jnp.float32),
            pltpu.VMEM((_C, _H), jnp.float32),
            pltpu.VMEM((_C, _H), jnp.float32),
            pltpu.VMEM((_C, _H), jnp.float32),
            pltpu.VMEM((_C, _H), jnp.float32),
            pltpu.VMEM((_C, _H), jnp.float32),
            pltpu.VMEM((_C, _H), jnp.float32),
            pltpu.VMEM((_ZC, _H), jnp.float32),
            pltpu.SemaphoreType.DMA,
            pltpu.SemaphoreType.DMA,
            pltpu.SemaphoreType.DMA,
            pltpu.SemaphoreType.DMA,
            pltpu.SemaphoreType.DMA,
            pltpu.SemaphoreType.DMA,
            pltpu.SemaphoreType.DMA,
            pltpu.SemaphoreType.DMA,
            pltpu.SemaphoreType.DMA,
            pltpu.SemaphoreType.DMA,
            pltpu.SemaphoreType.DMA,
            pltpu.SemaphoreType.DMA,
            pltpu.SemaphoreType.DMA,
            pltpu.SemaphoreType.DMA,
            pltpu.SemaphoreType.DMA,
            pltpu.SemaphoreType.DMA,
        ],
    )
    return f(h, col, row, w)


def _prep_edges(edge_index, edge_weight):
    col = edge_index[1].astype(jnp.int32).reshape(_NT, _EPT)
    row = edge_index[0].astype(jnp.int32).reshape(_NT, _EPT)
    w = edge_weight.astype(jnp.float32).reshape(_NT, _EPT)
    pad = ((0, 0), (0, _EPAD - _EPT))
    col = jnp.pad(col, pad).reshape(_NT, _NCH, _C)
    row = jnp.pad(row, pad).reshape(_NT, _NCH, _C)
    w = jnp.pad(w, pad).reshape(_NT, _NCH, _C)
    return col, row, w


def kernel(x, hl1_edge_index, hl1_edge_weight, hl2_edge_index, hl2_edge_weight,
           W_in1, b_in1, fW1, W_in2, b_in2, fW2, W_out, b_out):
    xp = jnp.pad(x, ((0, _NP - _N), (0, 0)))
    h = _proj_in(xp, jnp.stack([W_in1, W_in2]),
                 jnp.stack([b_in1, b_in2])[:, None, :])

    c1, r1, w1 = _prep_edges(hl1_edge_index, hl1_edge_weight)
    c2, r2, w2 = _prep_edges(hl2_edge_index, hl2_edge_weight)
    col = jnp.stack([c1, c2])
    row = jnp.stack([r1, r2])
    w = jnp.stack([w1, w2])

    ws = _sc_prop(h, col, row, w)
    ws = ws.reshape(_K + 1, 2, _NP, _H)

    fw = jnp.stack([
        jnp.pad(fW1, (0, 16 - (_K + 1)), constant_values=-1e30),
        jnp.pad(fW2, (0, 16 - (_K + 1)), constant_values=-1e30),
    ])
    return _epilogue(ws, fw, W_out, b_out)
